# Initial kernel scaffold; baseline (speedup 1.0000x reference)
import jax
import jax.numpy as jnp
from jax.experimental import pallas as pl


def kernel(x, edge_index, edge_attr, batch, We1, be1, W1, b1, g1, bt1, eps1, Wes, bes, Ws, bs, gs, bts, epss, Wc1, bc1, Wcls, bcls, Wf, bf):
    raise NotImplementedError("write your pallas kernel here")


# speedup vs baseline: 1.5313x; 1.5313x over previous
import functools
import jax
import jax.numpy as jnp
from jax import lax
from jax.experimental import pallas as pl
from jax.experimental.pallas import tpu as pltpu
from jax.experimental.pallas import tpu_sc as plsc

N = 10000
E = 320000
D = 128
ED = 16
G = 64
L = 3
H = 256
BN_EPS = 128.0

BN_ROWS = 1000
NBLK = N // BN_ROWS
BE = 4000
NW = 32
EPW = E // NW
CH = 80
NCH = EPW // CH
ZR = 208
RPT = 624


def _leaky(v):
    return jnp.where(v >= 0, v, 0.01 * v)


def _leaky2(v):
    return jnp.where(v >= 0, v, 1e-4 * v)


def _eproj_body(ea, W, b, out):
    out[0] = ea[...] @ W[0] + b[0]


def _eproj(ea, Wall, ball):
    return pl.pallas_call(
        _eproj_body,
        grid=(4, E // BE),
        in_specs=[
            pl.BlockSpec((BE, ED), lambda l, e: (e, 0)),
            pl.BlockSpec((1, ED, D), lambda l, e: (l, 0, 0)),
            pl.BlockSpec((1, 1, D), lambda l, e: (l, 0, 0)),
        ],
        out_specs=pl.BlockSpec((1, BE, D), lambda l, e: (l, e, 0)),
        out_shape=jax.ShapeDtypeStruct((4, E, D), jnp.float32),
    )(ea, Wall, ball.reshape(4, 1, D))


def _make_sc_edge(layer, apply_act):
    mesh = plsc.VectorSubcoreMesh(core_axis_name="c", subcore_axis_name="s")

    @functools.partial(
        pl.kernel,
        mesh=mesh,
        out_type=jax.ShapeDtypeStruct((2, N, D), jnp.float32),
        scratch_types=[
            pltpu.VMEM_SHARED((N, D), jnp.float32),
            pltpu.VMEM((CH, D), jnp.float32),
            pltpu.VMEM((CH, D), jnp.float32),
            pltpu.VMEM((CH,), jnp.int32),
            pltpu.VMEM((CH,), jnp.int32),
            pltpu.VMEM((ZR, D), jnp.float32),
            pltpu.VMEM((2, D), jnp.float32),
            pltpu.SemaphoreType.DMA,
            pltpu.SemaphoreType.DMA,
            pltpu.SemaphoreType.DMA,
            pltpu.SemaphoreType.DMA,
        ],
    )
    def k(h_hbm, ep_hbm, ei_hbm, ab_hbm, out_hbm,
          agg_sh, epb, xrb, srcb, dstb, zb, abb, s1, s2, s3, s4):
        cid = lax.axis_index("c")
        sid = lax.axis_index("s")
        wid = cid * 16 + sid

        def zrow(r, _):
            for j in range(D // 16):
                zb[r, pl.ds(j * 16, 16)] = jnp.zeros((16,), jnp.float32)
            return 0
        lax.fori_loop(0, ZR, zrow, 0)
        for kk in range(RPT // ZR):
            pltpu.sync_copy(zb, agg_sh.at[pl.ds(sid * RPT + kk * ZR, ZR)])

        @pl.when(sid == 15)
        def _():
            pltpu.sync_copy(zb.at[pl.ds(0, 16)], agg_sh.at[pl.ds(16 * RPT, 16)])
        pltpu.sync_copy(ab_hbm, abb)
        plsc.subcore_barrier()

        def chunk(it, _):
            base = wid * EPW + it * CH
            c_src = pltpu.async_copy(ei_hbm.at[pl.ds(base, CH)], srcb, s1)
            c_ep = pltpu.async_copy(ep_hbm.at[layer, pl.ds(base, CH)], epb, s2)
            c_dst = pltpu.async_copy(ei_hbm.at[pl.ds(E + base, CH)], dstb, s3)
            c_src.wait()
            c_g = pltpu.async_copy(h_hbm.at[srcb], xrb, s4)
            c_ep.wait()
            c_g.wait()

            def edge(e, _):
                for j in range(D // 16):
                    sl = pl.ds(j * 16, 16)
                    v = xrb[e, sl]
                    if apply_act:
                        v = abb[0, sl] * v + abb[1, sl]
                        v = _leaky2(v)
                    epb[e, sl] = jnp.maximum(v + epb[e, sl], 0.0)
                return 0
            lax.fori_loop(0, CH, edge, 0)
            c_dst.wait()
            pltpu.sync_copy(epb, agg_sh.at[dstb], add=True)
            return 0
        lax.fori_loop(0, NCH, chunk, 0)

        plsc.subcore_barrier()
        pltpu.sync_copy(agg_sh.at[pl.ds(sid * RPT, RPT)],
                        out_hbm.at[cid, pl.ds(sid * RPT, RPT)])

        @pl.when(sid == 15)
        def _():
            pltpu.sync_copy(agg_sh.at[pl.ds(16 * RPT, 16)],
                            out_hbm.at[cid, pl.ds(16 * RPT, 16)])

    return k


def _make_dense(apply_act):
    def body(hprev, ab, agg, W, b, eps, hpre_ref, stats_ref):
        i = pl.program_id(0)
        hin = hprev[...]
        if apply_act:
            hin = _leaky2(hin * ab[0:1, :] + ab[1:2, :])
        hs = hin * (1.0 + eps[0, 0]) + agg[0] + agg[1]
        hp = hs @ W[...] + b[...]
        hpre_ref[...] = hp
        st = jnp.stack([jnp.sum(hp, 0, keepdims=True),
                        jnp.sum(hp * hp, 0, keepdims=True)])

        @pl.when(i == 0)
        def _():
            stats_ref[...] = st

        @pl.when(i > 0)
        def _():
            stats_ref[...] += st

    def call(hprev, ab, agg, W, b, eps):
        return pl.pallas_call(
            body,
            grid=(NBLK,),
            in_specs=[
                pl.BlockSpec((BN_ROWS, D), lambda i: (i, 0)),
                pl.BlockSpec((2, D), lambda i: (0, 0)),
                pl.BlockSpec((2, BN_ROWS, D), lambda i: (0, i, 0)),
                pl.BlockSpec((D, D), lambda i: (0, 0)),
                pl.BlockSpec((1, D), lambda i: (0, 0)),
                pl.BlockSpec((1, 1), lambda i: (0, 0), memory_space=pltpu.SMEM),
            ],
            out_specs=[
                pl.BlockSpec((BN_ROWS, D), lambda i: (i, 0)),
                pl.BlockSpec((2, 1, D), lambda i: (0, 0, 0)),
            ],
            out_shape=[
                jax.ShapeDtypeStruct((N, D), jnp.float32),
                jax.ShapeDtypeStruct((2, 1, D), jnp.float32),
            ],
        )(hprev, ab, agg, W, b.reshape(1, D), eps.reshape(1, 1))
    return call


def _bn_affine(stats, g, bt):
    m = stats[0, 0] / N
    var = stats[1, 0] / N - m * m
    a = g / jnp.sqrt(var + BN_EPS)
    return jnp.stack([a, bt - m * a])


def _pool_body(hpre, ab, bf32, out_ref):
    i = pl.program_id(0)
    h4 = _leaky2(hpre[...] * ab[0:1, :] + ab[1:2, :])
    b = bf32[0][0]
    oh = (b[:, None] ==
          lax.broadcasted_iota(jnp.int32, (BN_ROWS, G), 1).astype(jnp.float32)
          ).astype(jnp.float32)
    part = lax.dot_general(oh, h4, (((0,), (0,)), ((), ())))

    @pl.when(i == 0)
    def _():
        out_ref[...] = part

    @pl.when(i > 0)
    def _():
        out_ref[...] += part


def _pool(hpre3, ab3, bf32):
    return pl.pallas_call(
        _pool_body,
        grid=(NBLK,),
        in_specs=[
            pl.BlockSpec((BN_ROWS, D), lambda i: (i, 0)),
            pl.BlockSpec((2, D), lambda i: (0, 0)),
            pl.BlockSpec((1, 1, BN_ROWS), lambda i: (i, 0, 0)),
        ],
        out_specs=pl.BlockSpec((G, D), lambda i: (0, 0)),
        out_shape=jax.ShapeDtypeStruct((G, D), jnp.float32),
    )(hpre3, ab3, bf32)


def _head_body(h1, h2, h3, h4, Aall, Ball, bf32, hpool,
               Wc1, bc1, Wcls, bcls, Wf, bf, o_ref):
    b = bf32[0][0]
    oh = (b[:, None] ==
          lax.broadcasted_iota(jnp.int32, (BN_ROWS, G), 1).astype(jnp.float32)
          ).astype(jnp.float32)
    hp = oh @ hpool[...]
    hs = [h1[...], h2[...], h3[...], h4[...]]
    hs = [_leaky2(h * Aall[i:i + 1, :] + Ball[i:i + 1, :])
          for i, h in enumerate(hs)]
    z = jnp.concatenate(hs + [hp], axis=1)
    a = z @ Wc1[...] + bc1[...]
    a = _leaky(a @ Wcls[0] + bcls[0:1, :])
    a = _leaky(a @ Wcls[1] + bcls[1:2, :])
    o = jnp.sum(a * Wf[...], axis=1, keepdims=True) + bf[0, 0]
    o_ref[...] = jax.nn.sigmoid(o)


def _head(hpres, Aall, Ball, bf32, hpool, Wc1, bc1, Wcls, bcls, Wf, bf):
    row = lambda i: (i, 0)
    full = lambda i: (0, 0)
    return pl.pallas_call(
        _head_body,
        grid=(NBLK,),
        in_specs=[
            pl.BlockSpec((BN_ROWS, D), row),
            pl.BlockSpec((BN_ROWS, D), row),
            pl.BlockSpec((BN_ROWS, D), row),
            pl.BlockSpec((BN_ROWS, D), row),
            pl.BlockSpec((4, D), full),
            pl.BlockSpec((4, D), full),
            pl.BlockSpec((1, 1, BN_ROWS), lambda i: (i, 0, 0)),
            pl.BlockSpec((G, D), full),
            pl.BlockSpec((D * (L + 2), H), full),
            pl.BlockSpec((1, H), full),
            pl.BlockSpec((2, H, H), lambda i: (0, 0, 0)),
            pl.BlockSpec((2, H), full),
            pl.BlockSpec((1, H), full),
            pl.BlockSpec((1, 1), full, memory_space=pltpu.SMEM),
        ],
        out_specs=pl.BlockSpec((BN_ROWS, 1), row),
        out_shape=jax.ShapeDtypeStruct((N, 1), jnp.float32),
    )(hpres[0], hpres[1], hpres[2], hpres[3], Aall, Ball, bf32, hpool,
      Wc1, bc1.reshape(1, H), Wcls, bcls, Wf.reshape(1, H), bf.reshape(1, 1))


def kernel(x, edge_index, edge_attr, batch, We1, be1, W1, b1, g1, bt1, eps1,
           Wes, bes, Ws, bs, gs, bts, epss, Wc1, bc1, Wcls, bcls, Wf, bf):
    Wall = jnp.concatenate([We1[None], Wes], axis=0)
    ball = jnp.concatenate([be1[None], bes], axis=0)
    EP = _eproj(edge_attr, Wall, ball)

    sc0 = _make_sc_edge(0, False)
    dense0 = _make_dense(False)
    dense1 = _make_dense(True)

    ab_id = jnp.stack([jnp.ones((D,), jnp.float32), jnp.zeros((D,), jnp.float32)])
    ei_flat = edge_index.reshape(-1)
    agg = sc0(x, EP, ei_flat, ab_id)
    hpre, stats = dense0(x, ab_id, agg, W1, b1, eps1)
    ab = _bn_affine(stats, g1, bt1)

    hpres = [hpre]
    abs_ = [ab]
    for i in range(L):
        sc_i = _make_sc_edge(i + 1, True)
        agg = sc_i(hpre, EP, ei_flat, ab)
        hpre, stats = dense1(hpre, ab, agg, Ws[i], bs[i], epss[i])
        ab = _bn_affine(stats, gs[i], bts[i])
        hpres.append(hpre)
        abs_.append(ab)

    bf32 = batch.astype(jnp.float32).reshape(NBLK, 1, BN_ROWS)
    hpool = _pool(hpres[3], abs_[3], bf32)
    Aall = jnp.stack([a[0] for a in abs_])
    Ball = jnp.stack([a[1] for a in abs_])
    return _head(hpres, Aall, Ball, bf32, hpool, Wc1, bc1, Wcls, bcls, Wf, bf)


# speedup vs baseline: 2.8635x; 1.8700x over previous
import functools
import jax
import jax.numpy as jnp
from jax import lax
from jax.experimental import pallas as pl
from jax.experimental.pallas import tpu as pltpu
from jax.experimental.pallas import tpu_sc as plsc

N = 10000
E = 320000
D = 128
ED = 16
G = 64
L = 3
H = 256
BN_EPS = 128.0

BN_ROWS = 1000
NBLK = N // BN_ROWS
BE = 4000
NW = 32
EPW = E // NW
CH = 80
NCH = EPW // CH
ZR = 208
RPT = 624


def _leaky(v):
    return jnp.where(v >= 0, v, 0.01 * v)


def _leaky2(v):
    return jnp.where(v >= 0, v, 1e-4 * v)


def _eproj_body(ea, W, b, out):
    out[0] = ea[...] @ W[0] + b[0]


def _eproj(ea, Wall, ball):
    return pl.pallas_call(
        _eproj_body,
        grid=(4, E // BE),
        in_specs=[
            pl.BlockSpec((BE, ED), lambda l, e: (e, 0)),
            pl.BlockSpec((1, ED, D), lambda l, e: (l, 0, 0)),
            pl.BlockSpec((1, 1, D), lambda l, e: (l, 0, 0)),
        ],
        out_specs=pl.BlockSpec((1, BE, D), lambda l, e: (l, e, 0)),
        out_shape=jax.ShapeDtypeStruct((4, E, D), jnp.float32),
    )(ea, Wall, ball.reshape(4, 1, D))


def _make_sc_edge(layer, apply_act):
    mesh = plsc.VectorSubcoreMesh(core_axis_name="c", subcore_axis_name="s")

    @functools.partial(
        pl.kernel,
        mesh=mesh,
        out_type=jax.ShapeDtypeStruct((2, N, D), jnp.float32),
        scratch_types=[
            pltpu.VMEM_SHARED((N, D), jnp.float32),
            pltpu.VMEM((CH, D), jnp.float32),
            pltpu.VMEM((CH, D), jnp.float32),
            pltpu.VMEM((CH,), jnp.int32),
            pltpu.VMEM((CH,), jnp.int32),
            pltpu.VMEM((ZR, D), jnp.float32),
            pltpu.VMEM((2, D), jnp.float32),
            pltpu.SemaphoreType.DMA,
            pltpu.SemaphoreType.DMA,
            pltpu.SemaphoreType.DMA,
            pltpu.SemaphoreType.DMA,
        ],
    )
    def k(h_hbm, ep_hbm, ei_hbm, ab_hbm, out_hbm,
          agg_sh, epb, xrb, srcb, dstb, zb, abb, s1, s2, s3, s4):
        cid = lax.axis_index("c")
        sid = lax.axis_index("s")
        wid = cid * 16 + sid

        def zrow(r, _):
            for j in range(D // 16):
                zb[r, pl.ds(j * 16, 16)] = jnp.zeros((16,), jnp.float32)
            return 0
        lax.fori_loop(0, ZR, zrow, 0)
        for kk in range(RPT // ZR):
            pltpu.sync_copy(zb, agg_sh.at[pl.ds(sid * RPT + kk * ZR, ZR)])

        @pl.when(sid == 15)
        def _():
            pltpu.sync_copy(zb.at[pl.ds(0, 16)], agg_sh.at[pl.ds(16 * RPT, 16)])
        pltpu.sync_copy(ab_hbm, abb)
        plsc.subcore_barrier()

        if apply_act:
            avs = [abb[0, pl.ds(j * 16, 16)] for j in range(D // 16)]
            bvs = [abb[1, pl.ds(j * 16, 16)] for j in range(D // 16)]

        def chunk(it, _):
            base = wid * EPW + it * CH
            c_src = pltpu.async_copy(ei_hbm.at[pl.ds(base, CH)], srcb, s1)
            c_ep = pltpu.async_copy(ep_hbm.at[layer, pl.ds(base, CH)], epb, s2)
            c_dst = pltpu.async_copy(ei_hbm.at[pl.ds(E + base, CH)], dstb, s3)
            c_src.wait()
            c_g = pltpu.async_copy(h_hbm.at[srcb], xrb, s4)
            c_ep.wait()
            c_g.wait()

            def edge(e, _):
                for j in range(D // 16):
                    sl = pl.ds(j * 16, 16)
                    v = xrb[e, sl]
                    if apply_act:
                        v = avs[j] * v + bvs[j]
                        v = _leaky2(v)
                    epb[e, sl] = jnp.maximum(v + epb[e, sl], 0.0)
                return 0
            lax.fori_loop(0, CH, edge, 0)
            c_dst.wait()
            pltpu.sync_copy(epb, agg_sh.at[dstb], add=True)
            return 0
        lax.fori_loop(0, NCH, chunk, 0)

        plsc.subcore_barrier()
        pltpu.sync_copy(agg_sh.at[pl.ds(sid * RPT, RPT)],
                        out_hbm.at[cid, pl.ds(sid * RPT, RPT)])

        @pl.when(sid == 15)
        def _():
            pltpu.sync_copy(agg_sh.at[pl.ds(16 * RPT, 16)],
                            out_hbm.at[cid, pl.ds(16 * RPT, 16)])

    return k


def _make_dense(apply_act):
    def body(hprev, ab, agg, W, b, eps, hpre_ref, stats_ref):
        i = pl.program_id(0)
        hin = hprev[...]
        if apply_act:
            hin = _leaky2(hin * ab[0:1, :] + ab[1:2, :])
        hs = hin * (1.0 + eps[0, 0]) + agg[0] + agg[1]
        hp = hs @ W[...] + b[...]
        hpre_ref[...] = hp
        st = jnp.stack([jnp.sum(hp, 0, keepdims=True),
                        jnp.sum(hp * hp, 0, keepdims=True)])

        @pl.when(i == 0)
        def _():
            stats_ref[...] = st

        @pl.when(i > 0)
        def _():
            stats_ref[...] += st

    def call(hprev, ab, agg, W, b, eps):
        return pl.pallas_call(
            body,
            grid=(NBLK,),
            in_specs=[
                pl.BlockSpec((BN_ROWS, D), lambda i: (i, 0)),
                pl.BlockSpec((2, D), lambda i: (0, 0)),
                pl.BlockSpec((2, BN_ROWS, D), lambda i: (0, i, 0)),
                pl.BlockSpec((D, D), lambda i: (0, 0)),
                pl.BlockSpec((1, D), lambda i: (0, 0)),
                pl.BlockSpec((1, 1), lambda i: (0, 0), memory_space=pltpu.SMEM),
            ],
            out_specs=[
                pl.BlockSpec((BN_ROWS, D), lambda i: (i, 0)),
                pl.BlockSpec((2, 1, D), lambda i: (0, 0, 0)),
            ],
            out_shape=[
                jax.ShapeDtypeStruct((N, D), jnp.float32),
                jax.ShapeDtypeStruct((2, 1, D), jnp.float32),
            ],
        )(hprev, ab, agg, W, b.reshape(1, D), eps.reshape(1, 1))
    return call


def _bn_affine(stats, g, bt):
    m = stats[0, 0] / N
    var = stats[1, 0] / N - m * m
    a = g / jnp.sqrt(var + BN_EPS)
    return jnp.stack([a, bt - m * a])


def _pool_body(hpre, ab, bf32, out_ref):
    i = pl.program_id(0)
    h4 = _leaky2(hpre[...] * ab[0:1, :] + ab[1:2, :])
    b = bf32[0][0]
    oh = (b[:, None] ==
          lax.broadcasted_iota(jnp.int32, (BN_ROWS, G), 1).astype(jnp.float32)
          ).astype(jnp.float32)
    part = lax.dot_general(oh, h4, (((0,), (0,)), ((), ())))

    @pl.when(i == 0)
    def _():
        out_ref[...] = part

    @pl.when(i > 0)
    def _():
        out_ref[...] += part


def _pool(hpre3, ab3, bf32):
    return pl.pallas_call(
        _pool_body,
        grid=(NBLK,),
        in_specs=[
            pl.BlockSpec((BN_ROWS, D), lambda i: (i, 0)),
            pl.BlockSpec((2, D), lambda i: (0, 0)),
            pl.BlockSpec((1, 1, BN_ROWS), lambda i: (i, 0, 0)),
        ],
        out_specs=pl.BlockSpec((G, D), lambda i: (0, 0)),
        out_shape=jax.ShapeDtypeStruct((G, D), jnp.float32),
    )(hpre3, ab3, bf32)


def _head_body(h1, h2, h3, h4, Aall, Ball, bf32, hpool,
               Wc1, bc1, Wcls, bcls, Wf, bf, o_ref):
    b = bf32[0][0]
    oh = (b[:, None] ==
          lax.broadcasted_iota(jnp.int32, (BN_ROWS, G), 1).astype(jnp.float32)
          ).astype(jnp.float32)
    hp = oh @ hpool[...]
    hs = [h1[...], h2[...], h3[...], h4[...]]
    hs = [_leaky2(h * Aall[i:i + 1, :] + Ball[i:i + 1, :])
          for i, h in enumerate(hs)]
    z = jnp.concatenate(hs + [hp], axis=1)
    a = z @ Wc1[...] + bc1[...]
    a = _leaky(a @ Wcls[0] + bcls[0:1, :])
    a = _leaky(a @ Wcls[1] + bcls[1:2, :])
    o = jnp.sum(a * Wf[...], axis=1, keepdims=True) + bf[0, 0]
    o_ref[...] = jax.nn.sigmoid(o)


def _head(hpres, Aall, Ball, bf32, hpool, Wc1, bc1, Wcls, bcls, Wf, bf):
    row = lambda i: (i, 0)
    full = lambda i: (0, 0)
    return pl.pallas_call(
        _head_body,
        grid=(NBLK,),
        in_specs=[
            pl.BlockSpec((BN_ROWS, D), row),
            pl.BlockSpec((BN_ROWS, D), row),
            pl.BlockSpec((BN_ROWS, D), row),
            pl.BlockSpec((BN_ROWS, D), row),
            pl.BlockSpec((4, D), full),
            pl.BlockSpec((4, D), full),
            pl.BlockSpec((1, 1, BN_ROWS), lambda i: (i, 0, 0)),
            pl.BlockSpec((G, D), full),
            pl.BlockSpec((D * (L + 2), H), full),
            pl.BlockSpec((1, H), full),
            pl.BlockSpec((2, H, H), lambda i: (0, 0, 0)),
            pl.BlockSpec((2, H), full),
            pl.BlockSpec((1, H), full),
            pl.BlockSpec((1, 1), full, memory_space=pltpu.SMEM),
        ],
        out_specs=pl.BlockSpec((BN_ROWS, 1), row),
        out_shape=jax.ShapeDtypeStruct((N, 1), jnp.float32),
    )(hpres[0], hpres[1], hpres[2], hpres[3], Aall, Ball, bf32, hpool,
      Wc1, bc1.reshape(1, H), Wcls, bcls, Wf.reshape(1, H), bf.reshape(1, 1))


def kernel(x, edge_index, edge_attr, batch, We1, be1, W1, b1, g1, bt1, eps1,
           Wes, bes, Ws, bs, gs, bts, epss, Wc1, bc1, Wcls, bcls, Wf, bf):
    Wall = jnp.concatenate([We1[None], Wes], axis=0)
    ball = jnp.concatenate([be1[None], bes], axis=0)
    EP = _eproj(edge_attr, Wall, ball)

    sc0 = _make_sc_edge(0, False)
    dense0 = _make_dense(False)
    dense1 = _make_dense(True)

    ab_id = jnp.stack([jnp.ones((D,), jnp.float32), jnp.zeros((D,), jnp.float32)])
    ei_flat = edge_index.reshape(-1)
    agg = sc0(x, EP, ei_flat, ab_id)
    hpre, stats = dense0(x, ab_id, agg, W1, b1, eps1)
    ab = _bn_affine(stats, g1, bt1)

    hpres = [hpre]
    abs_ = [ab]
    for i in range(L):
        sc_i = _make_sc_edge(i + 1, True)
        agg = sc_i(hpre, EP, ei_flat, ab)
        hpre, stats = dense1(hpre, ab, agg, Ws[i], bs[i], epss[i])
        ab = _bn_affine(stats, gs[i], bts[i])
        hpres.append(hpre)
        abs_.append(ab)

    bf32 = batch.astype(jnp.float32).reshape(NBLK, 1, BN_ROWS)
    hpool = _pool(hpres[3], abs_[3], bf32)
    Aall = jnp.stack([a[0] for a in abs_])
    Ball = jnp.stack([a[1] for a in abs_])
    return _head(hpres, Aall, Ball, bf32, hpool, Wc1, bc1, Wcls, bcls, Wf, bf)
